# baseline (speedup 1.0000x reference)
import functools

import jax
import jax.numpy as jnp
from jax import lax
from jax.experimental import pallas as pl
from jax.experimental.pallas import tpu as pltpu
from jax.experimental.pallas import tpu_sc as plsc

_TAU_INV = 10.0
_THRESH = 0.6
_BR = 256
_CPAD = 128
_L = 16


def _attn_stats_kernel(q_ref, k_ref, lb1h_ref, logit_ref,
                       cv_ref, scale_ref, idx_ref):
    i = pl.program_id(0)
    q = q_ref[...]
    k = k_ref[...]
    l = lax.dot_general(q, k, (((1,), (1,)), ((), ())),
                        preferred_element_type=jnp.float32) * _TAU_INV
    m = jnp.max(l, axis=1, keepdims=True)
    z = jnp.sum(jnp.exp(l - m), axis=1)
    idx = jnp.argmax(l, axis=1).astype(jnp.int32)
    p = 1.0 / z
    scale = jnp.where(p >= _THRESH, p, 0.0)

    num_lb = lb1h_ref.shape[0]
    n_lb_blocks = num_lb // _BR
    start = i * _BR
    lb_start = jnp.minimum(start, num_lb - _BR)
    ul_start = jnp.clip(start - num_lb, 0, logit_ref.shape[0] - _BR)
    lb_blk = lb1h_ref[pl.ds(lb_start, _BR), :]
    lg = logit_ref[pl.ds(ul_start, _BR), :]
    lm = jnp.max(lg, axis=1, keepdims=True)
    le = jnp.exp(lg - lm)
    ul_blk = le / jnp.sum(le, axis=1, keepdims=True)
    cv_blk = jnp.where(i < n_lb_blocks, lb_blk, ul_blk)
    pad = jnp.zeros((_BR, _CPAD - cv_blk.shape[1]), jnp.float32)
    cv_ref[...] = jnp.concatenate([cv_blk, pad], axis=1)

    scale_ref[...] = jnp.broadcast_to(scale[:, None], (_BR, _L))
    idx_ref[...] = idx[None, None, :]


def _tc_stats(anchor_feat, positive_feat, lb_one_hot, logits_x_ulb_1):
    N, d = anchor_feat.shape
    num_lb, C = lb_one_hot.shape
    num_ulb = logits_x_ulb_1.shape[0]
    nb = N // _BR
    cv16, scale16, idx3 = pl.pallas_call(
        _attn_stats_kernel,
        grid=(nb,),
        compiler_params=pltpu.CompilerParams(
            dimension_semantics=("parallel",)),
        in_specs=[
            pl.BlockSpec((_BR, d), lambda i: (i, 0)),
            pl.BlockSpec((N, d), lambda i: (0, 0)),
            pl.BlockSpec((num_lb, C), lambda i: (0, 0)),
            pl.BlockSpec((num_ulb, C), lambda i: (0, 0)),
        ],
        out_specs=[
            pl.BlockSpec((_BR, _CPAD), lambda i: (i, 0)),
            pl.BlockSpec((_BR, _L), lambda i: (i, 0)),
            pl.BlockSpec((1, 1, _BR), lambda i: (i, 0, 0)),
        ],
        out_shape=[
            jax.ShapeDtypeStruct((N, _CPAD), jnp.float32),
            jax.ShapeDtypeStruct((N, _L), jnp.float32),
            jax.ShapeDtypeStruct((nb, 1, _BR), jnp.int32),
        ],
    )(anchor_feat, positive_feat, lb_one_hot, logits_x_ulb_1)
    return cv16, scale16, idx3.reshape(N)


def _make_sc_gather(N):
    info = plsc.get_sparse_core_info()
    NC, NS = info.num_cores, info.num_subcores
    NW = NC * NS
    b_per_w = N // NW
    mesh = plsc.VectorSubcoreMesh(core_axis_name="c", subcore_axis_name="s")

    @functools.partial(
        pl.kernel, mesh=mesh,
        out_type=jax.ShapeDtypeStruct((N, _CPAD), jnp.float32),
        scratch_types=[
            pltpu.VMEM((b_per_w,), jnp.int32),
            pltpu.VMEM((b_per_w, _CPAD), jnp.float32),
            pltpu.VMEM((b_per_w, _L), jnp.float32),
            pltpu.SemaphoreType.DMA,
        ],
    )
    def gather_scale(cv_hbm, scale_hbm, idx_hbm, out_hbm,
                     idx_v, rows_v, scale_v, sem):
        wid = lax.axis_index("s") * NC + lax.axis_index("c")
        base = wid * b_per_w
        pltpu.sync_copy(idx_hbm.at[pl.ds(base, b_per_w)], idx_v)
        pltpu.async_copy(cv_hbm.at[idx_v], rows_v, sem).wait()
        pltpu.sync_copy(scale_hbm.at[pl.ds(base, b_per_w)], scale_v)

        def body(r, _):
            rows_v[r, pl.ds(0, _L)] = rows_v[r, pl.ds(0, _L)] * scale_v[r, :]
            return 0

        lax.fori_loop(0, b_per_w, body, 0)
        pltpu.sync_copy(rows_v, out_hbm.at[pl.ds(base, b_per_w)])

    return gather_scale


def kernel(anchor_feat, positive_feat, lb_feat, lb_one_hot, logits_x_lb,
           logits_x_ulb_1, logits_x_ulb_2, y_lb):
    N = anchor_feat.shape[0]
    num_lb = lb_one_hot.shape[0]
    C = lb_one_hot.shape[1]
    cv16, scale16, idx = _tc_stats(anchor_feat, positive_feat,
                                   lb_one_hot, logits_x_ulb_1)
    out16 = _make_sc_gather(N)(cv16, scale16, idx)
    out = out16[:, :C]
    return (anchor_feat, positive_feat, lb_feat, lb_one_hot,
            out[:num_lb], out[num_lb:], logits_x_ulb_2)


# speedup vs baseline: 1.0388x; 1.0388x over previous
import functools

import jax
import jax.numpy as jnp
from jax import lax
from jax.experimental import pallas as pl
from jax.experimental.pallas import tpu as pltpu
from jax.experimental.pallas import tpu_sc as plsc

_TAU_INV = 10.0
_THRESH = 0.6
_BR = 512
_CPAD = 128
_L = 16


def _attn_stats_kernel(q_ref, k_ref, lb1h_ref, logit_ref,
                       cv_ref, scale_ref, idx_ref):
    i = pl.program_id(0)
    q = q_ref[...]
    k = k_ref[...]
    l = lax.dot_general(q, k, (((1,), (1,)), ((), ())),
                        preferred_element_type=jnp.float32) * _TAU_INV
    m = jnp.max(l, axis=1, keepdims=True)
    z = jnp.sum(jnp.exp(l - m), axis=1)
    idx = jnp.argmax(l, axis=1).astype(jnp.int32)
    p = 1.0 / z
    scale = jnp.where(p >= _THRESH, p, 0.0)

    num_lb = lb1h_ref.shape[0]
    n_lb_blocks = num_lb // _BR
    start = i * _BR
    lb_start = jnp.minimum(start, num_lb - _BR)
    ul_start = jnp.clip(start - num_lb, 0, logit_ref.shape[0] - _BR)
    lb_blk = lb1h_ref[pl.ds(lb_start, _BR), :]
    lg = logit_ref[pl.ds(ul_start, _BR), :]
    lm = jnp.max(lg, axis=1, keepdims=True)
    le = jnp.exp(lg - lm)
    ul_blk = le / jnp.sum(le, axis=1, keepdims=True)
    cv_blk = jnp.where(i < n_lb_blocks, lb_blk, ul_blk)
    pad = jnp.zeros((_BR, _CPAD - cv_blk.shape[1]), jnp.float32)
    cv_ref[...] = jnp.concatenate([cv_blk, pad], axis=1)

    scale_ref[...] = jnp.broadcast_to(scale[:, None], (_BR, _L))
    idx_ref[...] = idx[None, None, :]


def _tc_stats(anchor_feat, positive_feat, lb_one_hot, logits_x_ulb_1):
    N, d = anchor_feat.shape
    num_lb, C = lb_one_hot.shape
    num_ulb = logits_x_ulb_1.shape[0]
    nb = N // _BR
    cv16, scale16, idx3 = pl.pallas_call(
        _attn_stats_kernel,
        grid=(nb,),
        compiler_params=pltpu.CompilerParams(
            dimension_semantics=("parallel",)),
        in_specs=[
            pl.BlockSpec((_BR, d), lambda i: (i, 0)),
            pl.BlockSpec((N, d), lambda i: (0, 0)),
            pl.BlockSpec((num_lb, C), lambda i: (0, 0)),
            pl.BlockSpec((num_ulb, C), lambda i: (0, 0)),
        ],
        out_specs=[
            pl.BlockSpec((_BR, _CPAD), lambda i: (i, 0)),
            pl.BlockSpec((_BR, _L), lambda i: (i, 0)),
            pl.BlockSpec((1, 1, _BR), lambda i: (i, 0, 0)),
        ],
        out_shape=[
            jax.ShapeDtypeStruct((N, _CPAD), jnp.float32),
            jax.ShapeDtypeStruct((N, _L), jnp.float32),
            jax.ShapeDtypeStruct((nb, 1, _BR), jnp.int32),
        ],
    )(anchor_feat, positive_feat, lb_one_hot, logits_x_ulb_1)
    return cv16, scale16, idx3.reshape(N)


def _make_sc_gather(N):
    info = plsc.get_sparse_core_info()
    NC, NS = info.num_cores, info.num_subcores
    NW = NC * NS
    b_per_w = N // NW
    mesh = plsc.VectorSubcoreMesh(core_axis_name="c", subcore_axis_name="s")

    @functools.partial(
        pl.kernel, mesh=mesh,
        out_type=jax.ShapeDtypeStruct((N, _CPAD), jnp.float32),
        scratch_types=[
            pltpu.VMEM((b_per_w,), jnp.int32),
            pltpu.VMEM((b_per_w, _CPAD), jnp.float32),
            pltpu.VMEM((b_per_w, _L), jnp.float32),
            pltpu.SemaphoreType.DMA,
        ],
    )
    def gather_scale(cv_hbm, scale_hbm, idx_hbm, out_hbm,
                     idx_v, rows_v, scale_v, sem):
        wid = lax.axis_index("s") * NC + lax.axis_index("c")
        base = wid * b_per_w
        pltpu.sync_copy(idx_hbm.at[pl.ds(base, b_per_w)], idx_v)
        pltpu.async_copy(cv_hbm.at[idx_v], rows_v, sem).wait()
        pltpu.sync_copy(scale_hbm.at[pl.ds(base, b_per_w)], scale_v)

        def body(r, _):
            rows_v[r, pl.ds(0, _L)] = rows_v[r, pl.ds(0, _L)] * scale_v[r, :]
            return 0

        lax.fori_loop(0, b_per_w, body, 0)
        pltpu.sync_copy(rows_v, out_hbm.at[pl.ds(base, b_per_w)])

    return gather_scale


def kernel(anchor_feat, positive_feat, lb_feat, lb_one_hot, logits_x_lb,
           logits_x_ulb_1, logits_x_ulb_2, y_lb):
    N = anchor_feat.shape[0]
    num_lb = lb_one_hot.shape[0]
    C = lb_one_hot.shape[1]
    cv16, scale16, idx = _tc_stats(anchor_feat, positive_feat,
                                   lb_one_hot, logits_x_ulb_1)
    out16 = _make_sc_gather(N)(cv16, scale16, idx)
    out = out16[:, :C]
    return (anchor_feat, positive_feat, lb_feat, lb_one_hot,
            out[:num_lb], out[num_lb:], logits_x_ulb_2)


# speedup vs baseline: 1.1144x; 1.0727x over previous
import functools

import jax
import jax.numpy as jnp
from jax import lax
from jax.experimental import pallas as pl
from jax.experimental.pallas import tpu as pltpu
from jax.experimental.pallas import tpu_sc as plsc

_TAU_INV = 10.0
_THRESH = 0.6
_BR = 512
_CPAD = 128
_L = 16


def _attn_stats_kernel(q_ref, k_ref, lb1h_ref, logit_ref,
                       cv_ref, scale_ref, idx_ref):
    i = pl.program_id(0)
    q = q_ref[...] * _TAU_INV
    k = k_ref[...]
    l = lax.dot_general(q, k, (((1,), (1,)), ((), ())),
                        preferred_element_type=jnp.float32)
    m = jnp.max(l, axis=1, keepdims=True)
    z = jnp.sum(jnp.exp(l - m), axis=1)
    idx = jnp.argmax(l, axis=1).astype(jnp.int32)
    p = 1.0 / z
    scale = jnp.where(p >= _THRESH, p, 0.0)

    num_lb = lb1h_ref.shape[0]
    n_lb_blocks = num_lb // _BR
    start = i * _BR
    lb_start = jnp.minimum(start, num_lb - _BR)
    ul_start = jnp.clip(start - num_lb, 0, logit_ref.shape[0] - _BR)
    lb_blk = lb1h_ref[pl.ds(lb_start, _BR), :]
    lg = logit_ref[pl.ds(ul_start, _BR), :]
    lm = jnp.max(lg, axis=1, keepdims=True)
    le = jnp.exp(lg - lm)
    ul_blk = le / jnp.sum(le, axis=1, keepdims=True)
    cv_blk = jnp.where(i < n_lb_blocks, lb_blk, ul_blk)
    pad = jnp.zeros((_BR, _CPAD - cv_blk.shape[1]), jnp.float32)
    cv_ref[...] = jnp.concatenate([cv_blk, pad], axis=1)

    scale_ref[...] = jnp.broadcast_to(scale[:, None], (_BR, _L))
    idx_ref[...] = idx[None, None, :]


def _tc_stats(anchor_feat, positive_feat, lb_one_hot, logits_x_ulb_1):
    N, d = anchor_feat.shape
    num_lb, C = lb_one_hot.shape
    num_ulb = logits_x_ulb_1.shape[0]
    nb = N // _BR
    cv16, scale16, idx3 = pl.pallas_call(
        _attn_stats_kernel,
        grid=(nb,),
        compiler_params=pltpu.CompilerParams(
            dimension_semantics=("parallel",)),
        in_specs=[
            pl.BlockSpec((_BR, d), lambda i: (i, 0)),
            pl.BlockSpec((N, d), lambda i: (0, 0)),
            pl.BlockSpec((num_lb, C), lambda i: (0, 0)),
            pl.BlockSpec((num_ulb, C), lambda i: (0, 0)),
        ],
        out_specs=[
            pl.BlockSpec((_BR, _CPAD), lambda i: (i, 0)),
            pl.BlockSpec((_BR, _L), lambda i: (i, 0)),
            pl.BlockSpec((1, 1, _BR), lambda i: (i, 0, 0)),
        ],
        out_shape=[
            jax.ShapeDtypeStruct((N, _CPAD), jnp.float32),
            jax.ShapeDtypeStruct((N, _L), jnp.float32),
            jax.ShapeDtypeStruct((nb, 1, _BR), jnp.int32),
        ],
    )(anchor_feat, positive_feat, lb_one_hot, logits_x_ulb_1)
    return cv16, scale16, idx3.reshape(N)


def _make_sc_gather(N):
    info = plsc.get_sparse_core_info()
    NC, NS = info.num_cores, info.num_subcores
    NW = NC * NS
    b_per_w = N // NW
    mesh = plsc.VectorSubcoreMesh(core_axis_name="c", subcore_axis_name="s")

    @functools.partial(
        pl.kernel, mesh=mesh,
        out_type=jax.ShapeDtypeStruct((N, _CPAD), jnp.float32),
        scratch_types=[
            pltpu.VMEM((b_per_w,), jnp.int32),
            pltpu.VMEM((b_per_w, _CPAD), jnp.float32),
            pltpu.VMEM((b_per_w, _L), jnp.float32),
            pltpu.SemaphoreType.DMA,
        ],
    )
    def gather_scale(cv_hbm, scale_hbm, idx_hbm, out_hbm,
                     idx_v, rows_v, scale_v, sem):
        wid = lax.axis_index("s") * NC + lax.axis_index("c")
        base = wid * b_per_w
        pltpu.sync_copy(idx_hbm.at[pl.ds(base, b_per_w)], idx_v)
        pltpu.async_copy(cv_hbm.at[idx_v], rows_v, sem).wait()
        pltpu.sync_copy(scale_hbm.at[pl.ds(base, b_per_w)], scale_v)

        def body(r, _):
            rows_v[r, pl.ds(0, _L)] = rows_v[r, pl.ds(0, _L)] * scale_v[r, :]
            return 0

        lax.fori_loop(0, b_per_w, body, 0)
        pltpu.sync_copy(rows_v, out_hbm.at[pl.ds(base, b_per_w)])

    return gather_scale


def kernel(anchor_feat, positive_feat, lb_feat, lb_one_hot, logits_x_lb,
           logits_x_ulb_1, logits_x_ulb_2, y_lb):
    N = anchor_feat.shape[0]
    num_lb = lb_one_hot.shape[0]
    C = lb_one_hot.shape[1]
    cv16, scale16, idx = _tc_stats(anchor_feat, positive_feat,
                                   lb_one_hot, logits_x_ulb_1)
    out16 = _make_sc_gather(N)(cv16, scale16, idx)
    out = out16[:, :C]
    return (anchor_feat, positive_feat, lb_feat, lb_one_hot,
            out[:num_lb], out[num_lb:], logits_x_ulb_2)
